# Initial kernel scaffold; baseline (speedup 1.0000x reference)
import functools

import jax
import jax.numpy as jnp
from jax.experimental import pallas as pl
from jax.experimental.pallas import tpu as pltpu

NUM_EXPERTS = 64
TOP_K = 6
AUX_COEF = 0.001
Z_COEF = 0.001


def _router_body(hs_ref, gwt_ref, bias_ref,
                 rw_ref, se_ref, probs_ref, aux_ref,
                 sump_ref, cnt_ref, z_ref,
                 *, total_tokens):
    i = pl.program_id(0)
    n = pl.num_programs(0)

    @pl.when(i == 0)
    def _init():
        sump_ref[...] = jnp.zeros_like(sump_ref)
        cnt_ref[...] = jnp.zeros_like(cnt_ref)
        z_ref[0, 0] = 0.0

    x = hs_ref[...]
    logits = jnp.dot(x, gwt_ref[...],
                     preferred_element_type=jnp.float32) + bias_ref[...]
    mx = jnp.max(logits, axis=-1, keepdims=True)
    ex = jnp.exp(logits - mx)
    sex = jnp.sum(ex, axis=-1, keepdims=True)
    probs = ex / sex
    probs_ref[...] = probs

    lse = mx + jnp.log(sex)
    z_ref[0, 0] += jnp.sum(lse * lse)

    iota = jax.lax.broadcasted_iota(jnp.int32, probs.shape, 1)
    work = probs
    cols_w = []
    cols_i = []
    selmask = jnp.zeros_like(probs)
    for _ in range(TOP_K):
        m = jnp.max(work, axis=-1, keepdims=True)
        idx = jnp.min(jnp.where(work == m, iota, NUM_EXPERTS),
                      axis=-1, keepdims=True)
        hit = iota == idx
        selmask += hit.astype(probs.dtype)
        cols_w.append(m)
        cols_i.append(idx)
        work = jnp.where(hit, -1.0, work)
    w = jnp.concatenate(cols_w, axis=1)
    s = jnp.concatenate(cols_i, axis=1)
    rw_ref[...] = w / jnp.sum(w, axis=-1, keepdims=True)
    se_ref[...] = s

    sump_ref[...] += jnp.sum(probs, axis=0, keepdims=True)
    cnt_ref[...] += jnp.sum(selmask, axis=0, keepdims=True)

    @pl.when(i == n - 1)
    def _fin():
        tpe = cnt_ref[...]
        frac = tpe / (jnp.sum(tpe) + 1e-9)
        avgp = sump_ref[...] / total_tokens
        lbl = jnp.sum(frac * avgp) * NUM_EXPERTS
        z = z_ref[0, 0] / total_tokens
        aux_ref[0, 0] = AUX_COEF * lbl + Z_COEF * z


def kernel(hidden_states, pressure_bias, gate_weight):
    b, s, h = hidden_states.shape
    t = b * s
    hs2 = hidden_states.reshape(t, h)
    gwt = gate_weight.T
    bias2 = pressure_bias.reshape(1, NUM_EXPERTS)

    blk = 512
    grid = (t // blk,)

    body = functools.partial(_router_body, total_tokens=float(t))
    rw, se, probs, aux = pl.pallas_call(
        body,
        grid=grid,
        in_specs=[
            pl.BlockSpec((blk, h), lambda i: (i, 0)),
            pl.BlockSpec((h, NUM_EXPERTS), lambda i: (0, 0)),
            pl.BlockSpec((1, NUM_EXPERTS), lambda i: (0, 0)),
        ],
        out_specs=[
            pl.BlockSpec((blk, TOP_K), lambda i: (i, 0)),
            pl.BlockSpec((blk, TOP_K), lambda i: (i, 0)),
            pl.BlockSpec((blk, NUM_EXPERTS), lambda i: (i, 0)),
            pl.BlockSpec((1, 1), lambda i: (0, 0)),
        ],
        out_shape=[
            jax.ShapeDtypeStruct((t, TOP_K), jnp.float32),
            jax.ShapeDtypeStruct((t, TOP_K), jnp.int32),
            jax.ShapeDtypeStruct((t, NUM_EXPERTS), jnp.float32),
            jax.ShapeDtypeStruct((1, 1), jnp.float32),
        ],
        scratch_shapes=[
            pltpu.VMEM((1, NUM_EXPERTS), jnp.float32),
            pltpu.VMEM((1, NUM_EXPERTS), jnp.float32),
            pltpu.SMEM((1, 1), jnp.float32),
        ],
    )(hs2, gwt, bias2)

    return (rw.reshape(b, s, TOP_K),
            se.reshape(b, s, TOP_K),
            probs.reshape(b, s, NUM_EXPERTS),
            aux.reshape(()))


# speedup vs baseline: 1.1180x; 1.1180x over previous
import functools

import jax
import jax.numpy as jnp
from jax.experimental import pallas as pl
from jax.experimental.pallas import tpu as pltpu

NUM_EXPERTS = 64
TOP_K = 6
AUX_COEF = 0.001
Z_COEF = 0.001


def _router_body(hs_ref, gwt_ref, bias_ref,
                 rw_ref, se_ref, probs_ref, aux_ref,
                 sump_ref, cnt_ref, z_ref,
                 *, total_tokens):
    i = pl.program_id(0)
    n = pl.num_programs(0)

    @pl.when(i == 0)
    def _init():
        sump_ref[...] = jnp.zeros_like(sump_ref)
        cnt_ref[...] = jnp.zeros_like(cnt_ref)
        z_ref[0, 0] = 0.0

    x = hs_ref[...]
    logits = jnp.dot(x, gwt_ref[...],
                     preferred_element_type=jnp.float32) + bias_ref[...]
    mx = jnp.max(logits, axis=-1, keepdims=True)
    ex = jnp.exp(logits - mx)
    sex = jnp.sum(ex, axis=-1, keepdims=True)
    probs = ex / sex
    probs_ref[...] = probs

    lse = mx + jnp.log(sex)
    z_ref[0, 0] += jnp.sum(lse * lse)

    iota = jax.lax.broadcasted_iota(jnp.int32, probs.shape, 1)
    work = probs
    cols_w = []
    cols_i = []
    selmask = jnp.zeros_like(probs)
    for _ in range(TOP_K):
        m = jnp.max(work, axis=-1, keepdims=True)
        idx = jnp.min(jnp.where(work == m, iota, NUM_EXPERTS),
                      axis=-1, keepdims=True)
        hit = iota == idx
        selmask += hit.astype(probs.dtype)
        cols_w.append(m)
        cols_i.append(idx)
        work = jnp.where(hit, -1.0, work)
    w = jnp.concatenate(cols_w, axis=1)
    s = jnp.concatenate(cols_i, axis=1)
    rw_ref[...] = w / jnp.sum(w, axis=-1, keepdims=True)
    se_ref[...] = s

    sump_ref[...] += jnp.sum(probs, axis=0, keepdims=True)
    cnt_ref[...] += jnp.sum(selmask, axis=0, keepdims=True)

    @pl.when(i == n - 1)
    def _fin():
        tpe = cnt_ref[...]
        frac = tpe / (jnp.sum(tpe) + 1e-9)
        avgp = sump_ref[...] / total_tokens
        lbl = jnp.sum(frac * avgp) * NUM_EXPERTS
        z = z_ref[0, 0] / total_tokens
        aux_ref[...] = jnp.reshape(AUX_COEF * lbl + Z_COEF * z, (1, 1))


def kernel(hidden_states, pressure_bias, gate_weight):
    b, s, h = hidden_states.shape
    t = b * s
    hs2 = hidden_states.reshape(t, h)
    gwt = gate_weight.T
    bias2 = pressure_bias.reshape(1, NUM_EXPERTS)

    blk = 512
    grid = (t // blk,)

    body = functools.partial(_router_body, total_tokens=float(t))
    rw, se, probs, aux = pl.pallas_call(
        body,
        grid=grid,
        in_specs=[
            pl.BlockSpec((blk, h), lambda i: (i, 0)),
            pl.BlockSpec((h, NUM_EXPERTS), lambda i: (0, 0)),
            pl.BlockSpec((1, NUM_EXPERTS), lambda i: (0, 0)),
        ],
        out_specs=[
            pl.BlockSpec((blk, TOP_K), lambda i: (i, 0)),
            pl.BlockSpec((blk, TOP_K), lambda i: (i, 0)),
            pl.BlockSpec((blk, NUM_EXPERTS), lambda i: (i, 0)),
            pl.BlockSpec((1, 1), lambda i: (0, 0)),
        ],
        out_shape=[
            jax.ShapeDtypeStruct((t, TOP_K), jnp.float32),
            jax.ShapeDtypeStruct((t, TOP_K), jnp.int32),
            jax.ShapeDtypeStruct((t, NUM_EXPERTS), jnp.float32),
            jax.ShapeDtypeStruct((1, 1), jnp.float32),
        ],
        scratch_shapes=[
            pltpu.VMEM((1, NUM_EXPERTS), jnp.float32),
            pltpu.VMEM((1, NUM_EXPERTS), jnp.float32),
            pltpu.SMEM((1, 1), jnp.float32),
        ],
    )(hs2, gwt, bias2)

    return (rw.reshape(b, s, TOP_K),
            se.reshape(b, s, TOP_K),
            probs.reshape(b, s, NUM_EXPERTS),
            aux.reshape(()))


# speedup vs baseline: 1.1735x; 1.0496x over previous
import functools

import jax
import jax.numpy as jnp
from jax import lax
from jax.experimental import pallas as pl
from jax.experimental.pallas import tpu as pltpu
from jax.experimental.pallas import tpu_sc as plsc

NUM_EXPERTS = 64
TOP_K = 6
AUX_COEF = 0.001
Z_COEF = 0.001


def _dense_body(hs_ref, gwt_ref, bias_ref,
                probs_ref, sump_ref, z_ref,
                sump_acc, z_acc):
    i = pl.program_id(0)
    n = pl.num_programs(0)

    @pl.when(i == 0)
    def _init():
        sump_acc[...] = jnp.zeros_like(sump_acc)
        z_acc[0, 0] = 0.0

    x = hs_ref[...]
    logits = jnp.dot(x, gwt_ref[...],
                     preferred_element_type=jnp.float32) + bias_ref[...]
    mx = jnp.max(logits, axis=-1, keepdims=True)
    ex = jnp.exp(logits - mx)
    sex = jnp.sum(ex, axis=-1, keepdims=True)
    probs = ex / sex
    probs_ref[...] = probs

    lse = mx + jnp.log(sex)
    z_acc[0, 0] += jnp.sum(lse * lse)
    sump_acc[...] += jnp.sum(probs, axis=0, keepdims=True)

    @pl.when(i == n - 1)
    def _fin():
        sump_ref[...] = sump_acc[...]
        z_ref[...] = jnp.reshape(z_acc[0, 0], (1, 1))


def _dense_call(hs2, gwt, bias2, t, h, blk):
    return pl.pallas_call(
        _dense_body,
        grid=(t // blk,),
        in_specs=[
            pl.BlockSpec((blk, h), lambda i: (i, 0)),
            pl.BlockSpec((h, NUM_EXPERTS), lambda i: (0, 0)),
            pl.BlockSpec((1, NUM_EXPERTS), lambda i: (0, 0)),
        ],
        out_specs=[
            pl.BlockSpec((blk, NUM_EXPERTS), lambda i: (i, 0)),
            pl.BlockSpec((1, NUM_EXPERTS), lambda i: (0, 0)),
            pl.BlockSpec((1, 1), lambda i: (0, 0)),
        ],
        out_shape=[
            jax.ShapeDtypeStruct((t, NUM_EXPERTS), jnp.float32),
            jax.ShapeDtypeStruct((1, NUM_EXPERTS), jnp.float32),
            jax.ShapeDtypeStruct((1, 1), jnp.float32),
        ],
        scratch_shapes=[
            pltpu.VMEM((1, NUM_EXPERTS), jnp.float32),
            pltpu.SMEM((1, 1), jnp.float32),
        ],
    )(hs2, gwt, bias2)


def _make_sc_topk(t, total_tokens):
    NW = 32
    tpw = t // NW
    ch = 256
    mesh = plsc.VectorSubcoreMesh(core_axis_name="c", subcore_axis_name="s")

    @functools.partial(
        pl.kernel, mesh=mesh,
        compiler_params=pltpu.CompilerParams(needs_layout_passes=False),
        out_type=[
            jax.ShapeDtypeStruct((t, 16), jnp.float32),
            jax.ShapeDtypeStruct((t, 16), jnp.int32),
            jax.ShapeDtypeStruct((32, NUM_EXPERTS), jnp.float32),
        ],
        scratch_types=[
            pltpu.VMEM((ch, NUM_EXPERTS), jnp.float32),
            pltpu.VMEM((ch, 16), jnp.float32),
            pltpu.VMEM((ch, 16), jnp.int32),
            pltpu.VMEM((NUM_EXPERTS,), jnp.float32),
        ],
    )
    def sc_topk(probs_hbm, rw_hbm, se_hbm, hist_hbm,
                pv, rwv, sev, hist):
        c = lax.axis_index("c")
        s = lax.axis_index("s")
        wid = s * 2 + c
        base = wid * tpw
        lane = lax.broadcasted_iota(jnp.int32, (16,), 0)
        ones = jnp.ones((16,), jnp.float32)

        for j in range(4):
            hist[pl.ds(16 * j, 16)] = jnp.zeros((16,), jnp.float32)

        def tok_body(tok, carry):
            keys = []
            for j in range(4):
                row = pv[tok, pl.ds(16 * j, 16)]
                rb = lax.bitcast_convert_type(row, jnp.int32)
                kb = (rb & ~63) | ((63 - 16 * j) - lane)
                keys.append(lax.bitcast_convert_type(kb, jnp.float32))
            wrow = jnp.zeros((16,), jnp.float32)
            serow = jnp.zeros((16,), jnp.int32)
            for k in range(TOP_K):
                mm = jnp.maximum(jnp.maximum(keys[0], keys[1]),
                                 jnp.maximum(keys[2], keys[3]))
                mval = jnp.max(mm)
                mvec = jnp.full((16,), mval)
                mb = lax.bitcast_convert_type(mvec, jnp.int32)
                hitk = lane == k
                wrow = jnp.where(
                    hitk,
                    lax.bitcast_convert_type(mb & ~63, jnp.float32), wrow)
                serow = jnp.where(hitk, 63 - (mb & 63), serow)
                for j in range(4):
                    keys[j] = jnp.where(keys[j] == mvec, -1.0, keys[j])
            rwv[tok, pl.ds(0, 16)] = wrow / jnp.full((16,), jnp.sum(wrow))
            sev[tok, pl.ds(0, 16)] = serow
            plsc.addupdate_scatter(hist, [serow], ones, mask=lane < TOP_K)
            return carry

        for ci in range(tpw // ch):
            cb = base + ci * ch
            pltpu.sync_copy(probs_hbm.at[pl.ds(cb, ch)], pv)
            lax.fori_loop(0, ch, tok_body, 0)
            pltpu.sync_copy(rwv, rw_hbm.at[pl.ds(cb, ch)])
            pltpu.sync_copy(sev, se_hbm.at[pl.ds(cb, ch)])

        pltpu.sync_copy(hist, hist_hbm.at[wid])

    return sc_topk


def _aux_body(hist_ref, sump_ref, z_ref, aux_ref, *, total_tokens):
    cnt = jnp.sum(hist_ref[...], axis=0, keepdims=True)
    frac = cnt / (jnp.sum(cnt) + 1e-9)
    avgp = sump_ref[...] / total_tokens
    lbl = jnp.sum(frac * avgp) * NUM_EXPERTS
    aux_ref[...] = (AUX_COEF * jnp.reshape(lbl, (1, 1))
                    + Z_COEF * z_ref[...] / total_tokens)


def _aux_call(hist32, sump, z, total_tokens):
    body = functools.partial(_aux_body, total_tokens=total_tokens)
    return pl.pallas_call(
        body,
        out_shape=jax.ShapeDtypeStruct((1, 1), jnp.float32),
    )(hist32, sump, z)


def kernel(hidden_states, pressure_bias, gate_weight):
    b, s, h = hidden_states.shape
    t = b * s
    hs2 = hidden_states.reshape(t, h)
    gwt = gate_weight.T
    bias2 = pressure_bias.reshape(1, NUM_EXPERTS)

    probs, sump, z = _dense_call(hs2, gwt, bias2, t, h, blk=4096)

    rw16, se16, hist32 = _make_sc_topk(t, float(t))(probs)
    aux = _aux_call(hist32, sump, z, float(t))

    return (rw16[:, :TOP_K].reshape(b, s, TOP_K),
            se16[:, :TOP_K].reshape(b, s, TOP_K),
            probs.reshape(b, s, NUM_EXPERTS),
            aux.reshape(()))
